# Initial kernel scaffold; baseline (speedup 1.0000x reference)
import jax
import jax.numpy as jnp
from jax.experimental import pallas as pl


def kernel(memory, class_logits, geometry_logits):
    raise NotImplementedError("write your pallas kernel here")


# speedup vs baseline: 1.2872x; 1.2872x over previous
import functools

import jax
import jax.numpy as jnp
from jax import lax
from jax.experimental import pallas as pl
from jax.experimental.pallas import tpu as pltpu
from jax.experimental.pallas import tpu_sc as plsc

TOP_K_SEL = 300


def _scores_body(x_ref, o_ref):
    o_ref[0, 0, 0, :] = jnp.max(x_ref[...], axis=-1)[0]


def _scores(class_logits):
    b, n, c = class_logits.shape
    tn = 2000
    out = pl.pallas_call(
        _scores_body,
        grid=(b, n // tn),
        in_specs=[pl.BlockSpec((1, tn, c), lambda bi, i: (bi, i, 0))],
        out_specs=pl.BlockSpec((1, 1, 1, tn), lambda bi, i: (bi, i, 0, 0)),
        out_shape=jax.ShapeDtypeStruct((b, n // tn, 1, tn), jnp.float32),
    )(class_logits)
    return out.reshape(b, n)


def _sc_gather(idx_pad, mem2, log2, geo2, nw, npad):
    d1, d2, d3 = mem2.shape[1], log2.shape[1], geo2.shape[1]
    rows = idx_pad.shape[0]
    mesh = plsc.VectorSubcoreMesh(core_axis_name="c", subcore_axis_name="s")

    @functools.partial(
        pl.kernel,
        mesh=mesh,
        out_type=(
            jax.ShapeDtypeStruct((rows, d1), jnp.float32),
            jax.ShapeDtypeStruct((rows, d2), jnp.float32),
            jax.ShapeDtypeStruct((rows, d3), jnp.float32),
        ),
        scratch_types=[
            pltpu.VMEM((npad,), jnp.int32),
            pltpu.VMEM((npad, d1), jnp.float32),
            pltpu.VMEM((npad, d2), jnp.float32),
            pltpu.VMEM((npad, d3), jnp.float32),
            pltpu.SemaphoreType.DMA,
            pltpu.SemaphoreType.DMA,
            pltpu.SemaphoreType.DMA,
        ],
    )
    def k(idx_hbm, mem_hbm, log_hbm, geo_hbm, o1, o2, o3,
          idx_v, r1, r2, r3, s1, s2, s3):
        wid = lax.axis_index("s") * 2 + lax.axis_index("c")
        base = wid * npad
        pltpu.sync_copy(idx_hbm.at[pl.ds(base, npad)], idx_v)
        c1 = pltpu.async_copy(mem_hbm.at[idx_v], r1, s1)

        def fire(c, _):
            vec = idx_v[pl.ds(c * 16, 16)]
            for l in range(16):
                j = c * 16 + l
                row = vec[l]
                pltpu.async_copy(log_hbm.at[row], r2.at[j], s2)
                pltpu.async_copy(geo_hbm.at[row], r3.at[j], s3)
            return 0

        lax.fori_loop(0, npad // 16, fire, 0)

        def drain(j, _):
            pltpu.make_async_copy(log_hbm.at[0], r2.at[j], s2).wait()
            pltpu.make_async_copy(geo_hbm.at[0], r3.at[j], s3).wait()
            return 0

        lax.fori_loop(0, npad, drain, 0)
        c1.wait()
        pltpu.sync_copy(r1, o1.at[pl.ds(base, npad)])
        pltpu.sync_copy(r2, o2.at[pl.ds(base, npad)])
        pltpu.sync_copy(r3, o3.at[pl.ds(base, npad)])

    return k(idx_pad, mem2, log2, geo2)


def kernel(memory, class_logits, geometry_logits):
    b, n, d1 = memory.shape
    d2 = class_logits.shape[2]
    d3 = geometry_logits.shape[2]
    k = TOP_K_SEL

    scores = _scores(class_logits)
    _, topk_ind = lax.top_k(scores, k)

    info = plsc.get_sparse_core_info()
    nw = info.num_cores * info.num_subcores
    kw = (b * k) // nw
    npad = ((kw + 7) // 8) * 8
    flat = (topk_ind + (jnp.arange(b, dtype=jnp.int32) * n)[:, None]).astype(jnp.int32)
    flat = flat.reshape(nw, kw)
    padv = jnp.broadcast_to((jnp.arange(nw, dtype=jnp.int32) * 64)[:, None], (nw, npad - kw))
    idx_pad = jnp.concatenate([flat, padv], axis=1).reshape(nw * npad)

    mem2 = memory.reshape(b * n, d1)
    log2 = class_logits.reshape(b * n, d2)
    geo2 = geometry_logits.reshape(b * n, d3)

    o1, o2, o3 = _sc_gather(idx_pad, mem2, log2, geo2, nw, npad)
    o1 = o1.reshape(nw, npad, d1)[:, :kw].reshape(b, k, d1)
    o2 = o2.reshape(nw, npad, d2)[:, :kw].reshape(b, k, d2)
    o3 = o3.reshape(nw, npad, d3)[:, :kw].reshape(b, k, d3)
    return (o1, o2, o3)
